# baseline (speedup 1.0000x reference)
import functools

import jax
import jax.numpy as jnp
from jax import lax
from jax.experimental import pallas as pl
from jax.experimental.pallas import tpu as pltpu
from jax.experimental.pallas import tpu_sc as plsc

N_N = 10000
N_PAD = 10240
E_E = 160000
DF = 128
MM = 32
H1 = 768
EAW = 72
MLPD = 256
NB_G = 32
TE = 1000
TEE = 1000
TN = 1000
CH = 128
NW = 32


def _silu(x):
    return x / (1.0 + jnp.exp(-x))


def _ln_rows(x, g, b, eps=1e-5):
    mu = jnp.mean(x, axis=1, keepdims=True)
    var = jnp.mean((x - mu) ** 2, axis=1, keepdims=True)
    return (x - mu) * lax.rsqrt(var + eps) * g + b


def _sc_gather2(table, ia, ib):
    n, d = table.shape
    e = ia.shape[0]
    nch = e // CH
    mesh = plsc.VectorSubcoreMesh(core_axis_name="c", subcore_axis_name="s")

    @functools.partial(
        pl.kernel,
        out_type=(jax.ShapeDtypeStruct((e, d), table.dtype),
                  jax.ShapeDtypeStruct((e, d), table.dtype)),
        mesh=mesh,
        scratch_types=[
            pltpu.VMEM((CH,), jnp.int32),
            pltpu.VMEM((CH,), jnp.int32),
            pltpu.VMEM((CH, d), table.dtype),
            pltpu.VMEM((CH, d), table.dtype),
            pltpu.SemaphoreType.DMA,
            pltpu.SemaphoreType.DMA,
        ],
    )
    def k(tab_h, ia_h, ib_h, oa_h, ob_h, ia_v, ib_v, ra_v, rb_v, sa, sb):
        cid = lax.axis_index("c")
        sid = lax.axis_index("s")
        wid = sid * 2 + cid
        nmine = nch // NW + jnp.where(wid < (nch % NW), 1, 0)

        def body(j, carry):
            off = (j * NW + wid) * CH
            pltpu.sync_copy(ia_h.at[pl.ds(off, CH)], ia_v)
            pltpu.sync_copy(ib_h.at[pl.ds(off, CH)], ib_v)
            ca = pltpu.async_copy(tab_h.at[ia_v], ra_v, sa)
            cb = pltpu.async_copy(tab_h.at[ib_v], rb_v, sb)
            ca.wait()
            pltpu.sync_copy(ra_v, oa_h.at[pl.ds(off, CH)])
            cb.wait()
            pltpu.sync_copy(rb_v, ob_h.at[pl.ds(off, CH)])
            return carry

        lax.fori_loop(0, nmine, body, 0)

    return k(table, ia, ib)


def _sc_scatter(vals, idx):
    e, d = vals.shape
    nch = e // CH
    rpt = N_PAD // 16
    mesh = plsc.VectorSubcoreMesh(core_axis_name="c", subcore_axis_name="s")

    @functools.partial(
        pl.kernel,
        out_type=jax.ShapeDtypeStruct((2, N_PAD, d), jnp.float32),
        mesh=mesh,
        scratch_types=[
            pltpu.VMEM((CH,), jnp.int32),
            pltpu.VMEM((CH, d), jnp.float32),
            pltpu.VMEM_SHARED((N_PAD, d), jnp.float32),
            pltpu.SemaphoreType.DMA,
        ],
    )
    def k(v_h, i_h, o_h, idx_v, rows_v, acc, sem):
        cid = lax.axis_index("c")
        sid = lax.axis_index("s")
        wid = sid * 2 + cid
        z16 = jnp.zeros((16,), jnp.float32)

        def zb(i, carry):
            for c0 in range(d // 16):
                rows_v[i, pl.ds(c0 * 16, 16)] = z16
            return carry

        lax.fori_loop(0, CH, zb, 0)
        tbase = sid * rpt

        def za(kk, carry):
            pltpu.sync_copy(rows_v, acc.at[pl.ds(tbase + kk * CH, CH)])
            return carry

        lax.fori_loop(0, rpt // CH, za, 0)
        plsc.subcore_barrier()

        nmine = nch // NW + jnp.where(wid < (nch % NW), 1, 0)

        def body(j, carry):
            off = (j * NW + wid) * CH
            pltpu.sync_copy(i_h.at[pl.ds(off, CH)], idx_v)
            pltpu.sync_copy(v_h.at[pl.ds(off, CH)], rows_v)
            pltpu.sync_copy(rows_v, acc.at[idx_v], add=True)
            return carry

        lax.fori_loop(0, nmine, body, 0)
        plsc.subcore_barrier()

        def wb(kk, carry):
            o = tbase + kk * CH
            pltpu.sync_copy(acc.at[pl.ds(o, CH)], o_h.at[cid, pl.ds(o, CH)])
            return carry

        lax.fori_loop(0, rpt // CH, wb, 0)

    return k(vals, idx)


def _sc_count(idx, e):
    d = 16
    nch = e // CH
    rpt = N_PAD // 16
    mesh = plsc.VectorSubcoreMesh(core_axis_name="c", subcore_axis_name="s")

    @functools.partial(
        pl.kernel,
        out_type=jax.ShapeDtypeStruct((2, N_PAD, d), jnp.float32),
        mesh=mesh,
        scratch_types=[
            pltpu.VMEM((CH,), jnp.int32),
            pltpu.VMEM((CH, d), jnp.float32),
            pltpu.VMEM((CH, d), jnp.float32),
            pltpu.VMEM_SHARED((N_PAD, d), jnp.float32),
            pltpu.SemaphoreType.DMA,
        ],
    )
    def k(i_h, o_h, idx_v, ones_v, zer_v, acc, sem):
        cid = lax.axis_index("c")
        sid = lax.axis_index("s")
        wid = sid * 2 + cid
        z16 = jnp.zeros((16,), jnp.float32)
        o16 = jnp.full((16,), 1.0, jnp.float32)

        def zb(i, carry):
            zer_v[i, pl.ds(0, 16)] = z16
            ones_v[i, pl.ds(0, 16)] = o16
            return carry

        lax.fori_loop(0, CH, zb, 0)
        tbase = sid * rpt

        def za(kk, carry):
            pltpu.sync_copy(zer_v, acc.at[pl.ds(tbase + kk * CH, CH)])
            return carry

        lax.fori_loop(0, rpt // CH, za, 0)
        plsc.subcore_barrier()

        nmine = nch // NW + jnp.where(wid < (nch % NW), 1, 0)

        def body(j, carry):
            off = (j * NW + wid) * CH
            pltpu.sync_copy(i_h.at[pl.ds(off, CH)], idx_v)
            pltpu.sync_copy(ones_v, acc.at[idx_v], add=True)
            return carry

        lax.fori_loop(0, nmine, body, 0)
        plsc.subcore_barrier()

        def wb(kk, carry):
            o = tbase + kk * CH
            pltpu.sync_copy(acc.at[pl.ds(o, CH)], o_h.at[cid, pl.ds(o, CH)])
            return carry

        lax.fori_loop(0, rpt // CH, wb, 0)

    return k(idx)


def _tc_embed(ids2d, emb_p):
    grid = N_N // TN

    def body(ids_ref, emb_ref, o_ref):
        ids = ids_ref[...]
        cols = lax.broadcasted_iota(jnp.int32, (TN, 16), 1)
        oh = (cols == ids).astype(jnp.float32)
        o_ref[...] = jnp.dot(oh, emb_ref[...],
                             preferred_element_type=jnp.float32)

    return pl.pallas_call(
        body,
        grid=(grid,),
        in_specs=[pl.BlockSpec((TN, 1), lambda i: (i, 0)),
                  pl.BlockSpec((16, DF), lambda i: (0, 0))],
        out_specs=pl.BlockSpec((TN, DF), lambda i: (i, 0)),
        out_shape=jax.ShapeDtypeStruct((N_N, DF), jnp.float32),
    )(ids2d, emb_p)


def _tc_fourier(csrc, cdst, inv_sc):
    grid = E_E // TE

    def body(a_ref, b_ref, s_ref, o_ref):
        rel = a_ref[...] - b_ref[...]
        rd = jnp.sum(rel * rel, axis=1, keepdims=True)
        xs = rd * s_ref[0:1, :]
        sn = jnp.sin(xs)
        cs = jnp.cos(xs)
        pad = jnp.zeros((TE, EAW - 65), jnp.float32)
        o_ref[...] = jnp.concatenate([sn, cs, rd, pad],
                                     axis=1).astype(jnp.bfloat16)

    return pl.pallas_call(
        body,
        grid=(grid,),
        in_specs=[pl.BlockSpec((TE, DF), lambda i: (i, 0)),
                  pl.BlockSpec((TE, DF), lambda i: (i, 0)),
                  pl.BlockSpec((8, 32), lambda i: (0, 0))],
        out_specs=pl.BlockSpec((TE, EAW), lambda i: (i, 0)),
        out_shape=jax.ShapeDtypeStruct((E_E, EAW), jnp.bfloat16),
    )(csrc, cdst, inv_sc)


def _tc_edge1(xi, xj, csrc, cdst, inv_sc, w1c, b1p, w2p, b2p, gp, bp):
    grid = E_E // TEE

    def body(xi_ref, xj_ref, a_ref, b_ref, s_ref, w1_ref, b1_ref,
             w2_ref, b2_ref, g_ref, bb_ref, o_ref, ea_ref):
        rel = a_ref[...] - b_ref[...]
        rd = jnp.sum(rel * rel, axis=1, keepdims=True)
        xs = rd * s_ref[0:1, :]
        ea = jnp.concatenate(
            [jnp.sin(xs), jnp.cos(xs), rd,
             jnp.zeros((TEE, EAW - 65), jnp.float32)], axis=1)
        ea_ref[...] = ea.astype(jnp.bfloat16)
        x = jnp.concatenate(
            [xi_ref[...], xj_ref[...], ea], axis=1).astype(jnp.bfloat16)
        z = jnp.dot(x, w1_ref[...], preferred_element_type=jnp.float32)
        z = z + b1_ref[0:1, :]
        m = _silu(z).astype(jnp.bfloat16)
        y = jnp.dot(m, w2_ref[...], preferred_element_type=jnp.float32)
        y = _silu(y + b2_ref[0:1, :])
        o_ref[...] = _ln_rows(y, g_ref[0:1, :], bb_ref[0:1, :])

    return pl.pallas_call(
        body,
        grid=(grid,),
        in_specs=[
            pl.BlockSpec((TEE, DF), lambda i: (i, 0)),
            pl.BlockSpec((TEE, DF), lambda i: (i, 0)),
            pl.BlockSpec((TEE, DF), lambda i: (i, 0)),
            pl.BlockSpec((TEE, DF), lambda i: (i, 0)),
            pl.BlockSpec((8, 32), lambda i: (0, 0)),
            pl.BlockSpec((2 * DF + EAW, H1), lambda i: (0, 0)),
            pl.BlockSpec((8, H1), lambda i: (0, 0)),
            pl.BlockSpec((H1, MM), lambda i: (0, 0)),
            pl.BlockSpec((8, MM), lambda i: (0, 0)),
            pl.BlockSpec((8, MM), lambda i: (0, 0)),
            pl.BlockSpec((8, MM), lambda i: (0, 0)),
        ],
        out_specs=(pl.BlockSpec((TEE, MM), lambda i: (i, 0)),
                   pl.BlockSpec((TEE, EAW), lambda i: (i, 0))),
        out_shape=(jax.ShapeDtypeStruct((E_E, MM), jnp.float32),
                   jax.ShapeDtypeStruct((E_E, EAW), jnp.bfloat16)),
    )(xi, xj, csrc, cdst, inv_sc, w1c, b1p, w2p, b2p, gp, bp)


def _tc_edge(xi, xj, ea, w1c, b1p, w2p, b2p, gp, bp):
    grid = E_E // TEE

    def body(xi_ref, xj_ref, ea_ref, w1_ref, b1_ref,
             w2_ref, b2_ref, g_ref, b_ref, o_ref):
        x = jnp.concatenate(
            [xi_ref[...], xj_ref[...],
             ea_ref[...].astype(jnp.float32)], axis=1).astype(jnp.bfloat16)
        z = jnp.dot(x, w1_ref[...], preferred_element_type=jnp.float32)
        z = z + b1_ref[0:1, :]
        m = _silu(z).astype(jnp.bfloat16)
        y = jnp.dot(m, w2_ref[...], preferred_element_type=jnp.float32)
        y = _silu(y + b2_ref[0:1, :])
        o_ref[...] = _ln_rows(y, g_ref[0:1, :], b_ref[0:1, :])

    return pl.pallas_call(
        body,
        grid=(grid,),
        in_specs=[
            pl.BlockSpec((TEE, DF), lambda i: (i, 0)),
            pl.BlockSpec((TEE, DF), lambda i: (i, 0)),
            pl.BlockSpec((TEE, EAW), lambda i: (i, 0)),
            pl.BlockSpec((2 * DF + EAW, H1), lambda i: (0, 0)),
            pl.BlockSpec((8, H1), lambda i: (0, 0)),
            pl.BlockSpec((H1, MM), lambda i: (0, 0)),
            pl.BlockSpec((8, MM), lambda i: (0, 0)),
            pl.BlockSpec((8, MM), lambda i: (0, 0)),
            pl.BlockSpec((8, MM), lambda i: (0, 0)),
        ],
        out_specs=pl.BlockSpec((TEE, MM), lambda i: (i, 0)),
        out_shape=jax.ShapeDtypeStruct((E_E, MM), jnp.float32),
    )(xi, xj, ea, w1c, b1p, w2p, b2p, gp, bp)


def _tc_node(feats, parts, cnts, w1h, w1m, nb1, w2t, nb2,
             eng, enb, n1g, n1b, n2g, n2b):
    grid = N_N // TN

    def body(f_ref, p_ref, c_ref, w1h_ref, w1m_ref, nb1_ref, w2t_ref,
             nb2_ref, eng_ref, enb_ref, n1g_ref, n1b_ref, n2g_ref,
             n2b_ref, o_ref):
        msum = p_ref[0] + p_ref[1]
        cnt = c_ref[0][:, 0:1] + c_ref[1][:, 0:1]
        m = msum / jnp.maximum(cnt, 1.0)
        m = _ln_rows(m, eng_ref[0:1, :], enb_ref[0:1, :])
        f = f_ref[...]
        h = _ln_rows(f, n1g_ref[0:1, :], n1b_ref[0:1, :])
        z = jnp.dot(h, w1h_ref[...], preferred_element_type=jnp.float32)
        z = z + jnp.dot(m, w1m_ref[...], preferred_element_type=jnp.float32)
        z = _silu(z + nb1_ref[0:1, :])
        h2 = jnp.dot(z, w2t_ref[...], preferred_element_type=jnp.float32)
        h2 = h2 + nb2_ref[0:1, :]
        h2 = _ln_rows(h2, n2g_ref[0:1, :], n2b_ref[0:1, :])
        o_ref[...] = f + h2

    return pl.pallas_call(
        body,
        grid=(grid,),
        in_specs=[
            pl.BlockSpec((TN, DF), lambda i: (i, 0)),
            pl.BlockSpec((2, TN, MM), lambda i: (0, i, 0)),
            pl.BlockSpec((2, TN, 16), lambda i: (0, i, 0)),
            pl.BlockSpec((DF, 2 * DF), lambda i: (0, 0)),
            pl.BlockSpec((MM, 2 * DF), lambda i: (0, 0)),
            pl.BlockSpec((8, 2 * DF), lambda i: (0, 0)),
            pl.BlockSpec((2 * DF, DF), lambda i: (0, 0)),
            pl.BlockSpec((8, DF), lambda i: (0, 0)),
            pl.BlockSpec((8, MM), lambda i: (0, 0)),
            pl.BlockSpec((8, MM), lambda i: (0, 0)),
            pl.BlockSpec((8, DF), lambda i: (0, 0)),
            pl.BlockSpec((8, DF), lambda i: (0, 0)),
            pl.BlockSpec((8, DF), lambda i: (0, 0)),
            pl.BlockSpec((8, DF), lambda i: (0, 0)),
        ],
        out_specs=pl.BlockSpec((TN, DF), lambda i: (i, 0)),
        out_shape=jax.ShapeDtypeStruct((N_N, DF), jnp.float32),
    )(feats, parts, cnts, w1h, w1m, nb1, w2t, nb2,
      eng, enb, n1g, n1b, n2g, n2b)


def _tc_readout(fl, batch3d, ws):
    (w1, b1, w2, b2, w3, b3, v1, c1, v2, c2, v3p, c3p) = ws
    grid = N_N // TN

    def body(f0_ref, f1_ref, f2_ref, f3_ref, f4_ref, f5_ref, bt_ref,
             w1_ref, b1_ref, w2_ref, b2_ref, w3_ref, b3_ref,
             v1_ref, c1_ref, v2_ref, c2_ref, v3_ref, c3_ref,
             o_ref, acc, cacc):
        i = pl.program_id(0)

        @pl.when(i == 0)
        def _():
            acc[...] = jnp.zeros_like(acc)
            cacc[...] = jnp.zeros_like(cacc)

        x = jnp.concatenate(
            [f0_ref[...], f1_ref[...], f2_ref[...],
             f3_ref[...], f4_ref[...], f5_ref[...]], axis=1)
        f = _silu(x)
        f = _silu(jnp.dot(f, w1_ref[...],
                          preferred_element_type=jnp.float32) + b1_ref[0:1, :])
        f = _silu(jnp.dot(f, w2_ref[...],
                          preferred_element_type=jnp.float32) + b2_ref[0:1, :])
        f = _silu(jnp.dot(f, w3_ref[...],
                          preferred_element_type=jnp.float32) + b3_ref[0:1, :])
        bt = bt_ref[0, 0, :]
        oh = (lax.broadcasted_iota(jnp.int32, (NB_G, TN), 0)
              == bt[None, :]).astype(jnp.float32)
        acc[...] = acc[...] + jnp.dot(oh, f,
                                      preferred_element_type=jnp.float32)
        cacc[...] = cacc[...] + jnp.sum(oh, axis=1, keepdims=True)

        @pl.when(i == grid - 1)
        def _():
            mean = acc[...] / jnp.maximum(cacc[:, 0:1], 1.0)
            g = _silu(jnp.dot(mean, v1_ref[...],
                              preferred_element_type=jnp.float32)
                      + c1_ref[0:1, :])
            g = _silu(jnp.dot(g, v2_ref[...],
                              preferred_element_type=jnp.float32)
                      + c2_ref[0:1, :])
            o_ref[...] = (jnp.dot(g, v3_ref[...],
                                  preferred_element_type=jnp.float32)
                          + c3_ref[0:1, :])

    return pl.pallas_call(
        body,
        grid=(grid,),
        in_specs=[
            pl.BlockSpec((TN, DF), lambda i: (i, 0)),
            pl.BlockSpec((TN, DF), lambda i: (i, 0)),
            pl.BlockSpec((TN, DF), lambda i: (i, 0)),
            pl.BlockSpec((TN, DF), lambda i: (i, 0)),
            pl.BlockSpec((TN, DF), lambda i: (i, 0)),
            pl.BlockSpec((TN, DF), lambda i: (i, 0)),
            pl.BlockSpec((1, 1, TN), lambda i: (i, 0, 0)),
            pl.BlockSpec((6 * DF, MLPD), lambda i: (0, 0)),
            pl.BlockSpec((8, MLPD), lambda i: (0, 0)),
            pl.BlockSpec((MLPD, MLPD), lambda i: (0, 0)),
            pl.BlockSpec((8, MLPD), lambda i: (0, 0)),
            pl.BlockSpec((MLPD, MLPD), lambda i: (0, 0)),
            pl.BlockSpec((8, MLPD), lambda i: (0, 0)),
            pl.BlockSpec((MLPD, MLPD), lambda i: (0, 0)),
            pl.BlockSpec((8, MLPD), lambda i: (0, 0)),
            pl.BlockSpec((MLPD, MLPD), lambda i: (0, 0)),
            pl.BlockSpec((8, MLPD), lambda i: (0, 0)),
            pl.BlockSpec((MLPD, DF), lambda i: (0, 0)),
            pl.BlockSpec((8, DF), lambda i: (0, 0)),
        ],
        out_specs=pl.BlockSpec((NB_G, DF), lambda i: (0, 0)),
        out_shape=jax.ShapeDtypeStruct((NB_G, DF), jnp.float32),
        scratch_shapes=[pltpu.VMEM((NB_G, MLPD), jnp.float32),
                        pltpu.VMEM((NB_G, DF), jnp.float32)],
    )(*fl, batch3d, w1, b1, w2, b2, w3, b3, v1, c1, v2, c2, v3p, c3p)


def _row8(v, width):
    out = jnp.zeros((8, width), jnp.float32)
    return out.at[0, :v.shape[0]].set(v)


def _prep_edge_weights(p):
    w1 = p["e_w1"]
    w1c = jnp.zeros((2 * DF + EAW, H1), jnp.float32).at[:321, :642].set(
        w1.T).astype(jnp.bfloat16)
    b1p = _row8(p["e_b1"], H1)
    w2p = jnp.zeros((H1, MM), jnp.float32).at[:642, :].set(
        p["e_w2"].T).astype(jnp.bfloat16)
    b2p = _row8(p["e_b2"], MM)
    gp = _row8(p["en_g"], MM)
    bp = _row8(p["en_b"], MM)
    return (w1c, b1p, w2p, b2p, gp, bp)


def _prep_node_weights(p):
    nw1 = p["n_w1"]
    w1h = nw1[:, :DF].T
    w1m = nw1[:, DF:].T
    nb1 = _row8(p["n_b1"], 2 * DF)
    w2t = p["n_w2"].T
    nb2 = _row8(p["n_b2"], DF)
    eng = _row8(p["en_g"], MM)
    enb = _row8(p["en_b"], MM)
    n1g = _row8(p["nn1_g"], DF)
    n1b = _row8(p["nn1_b"], DF)
    n2g = _row8(p["nn2_g"], DF)
    n2b = _row8(p["nn2_b"], DF)
    return (w1h, w1m, nb1, w2t, nb2, eng, enb, n1g, n1b, n2g, n2b)


def kernel(atomids, coords, edge_index, batch, params):
    src = edge_index[0].astype(jnp.int32)
    dst = edge_index[1].astype(jnp.int32)

    emb_p = jnp.zeros((16, DF), jnp.float32).at[:11].set(params["emb"])
    coords_p = jnp.zeros((N_N, DF), jnp.float32).at[:, :3].set(coords)
    inv_sc = jnp.zeros((8, 32), jnp.float32).at[0].set(
        jnp.exp2(-jnp.arange(32, dtype=jnp.float32)))

    csrc, cdst = _sc_gather2(coords_p, src, dst)
    feats = _tc_embed(atomids.reshape(N_N, 1).astype(jnp.int32), emb_p)
    cnts = _sc_count(dst, E_E)

    fl = [feats]
    ea = None
    for li, p in enumerate(params["kernels"]):
        ew = _prep_edge_weights(p)
        nw = _prep_node_weights(p)
        xi, xj = _sc_gather2(feats, dst, src)
        if li == 0:
            m2, ea = _tc_edge1(xi, xj, csrc, cdst, inv_sc, *ew)
        else:
            m2 = _tc_edge(xi, xj, ea, *ew)
        parts = _sc_scatter(m2, dst)
        feats = _tc_node(feats, parts, cnts, *nw)
        fl.append(feats)

    w1 = params["fnn"][0][0].T
    b1 = _row8(params["fnn"][0][1], MLPD)
    w2 = params["fnn"][1][0].T
    b2 = _row8(params["fnn"][1][1], MLPD)
    w3 = params["fnn"][2][0].T
    b3 = _row8(params["fnn"][2][1], MLPD)
    v1 = params["fnn2"][0][0].T
    c1 = _row8(params["fnn2"][0][1], MLPD)
    v2 = params["fnn2"][1][0].T
    c2 = _row8(params["fnn2"][1][1], MLPD)
    v3p = jnp.zeros((MLPD, DF), jnp.float32).at[:, 0].set(params["fnn2"][2][0][0])
    c3p = jnp.full((8, DF), params["fnn2"][2][1][0], jnp.float32)

    batch3d = batch.astype(jnp.int32).reshape(N_N // TN, 1, TN)
    out = _tc_readout(fl, batch3d,
                      (w1, b1, w2, b2, w3, b3, v1, c1, v2, c2, v3p, c3p))
    return out[:, :1]


# speedup vs baseline: 1.2106x; 1.2106x over previous
import functools

import jax
import jax.numpy as jnp
from jax import lax
from jax.experimental import pallas as pl
from jax.experimental.pallas import tpu as pltpu
from jax.experimental.pallas import tpu_sc as plsc

N_N = 10000
N_PAD = 10240
E_E = 160000
DF = 128
MM = 32
H1 = 768
EAW = 72
MLPD = 256
NB_G = 32
TE = 1000
TEE = 1000
TN = 1000
CH = 128
NW = 32


def _silu(x):
    return x / (1.0 + jnp.exp(-x))


def _ln_rows(x, g, b, eps=1e-5):
    mu = jnp.mean(x, axis=1, keepdims=True)
    var = jnp.mean((x - mu) ** 2, axis=1, keepdims=True)
    return (x - mu) * lax.rsqrt(var + eps) * g + b


def _sc_gather2(table, ia, ib, ebase=0, e=None):
    n, d = table.shape
    if e is None:
        e = ia.shape[0]
    nch = e // CH
    mesh = plsc.VectorSubcoreMesh(core_axis_name="c", subcore_axis_name="s")

    @functools.partial(
        pl.kernel,
        out_type=(jax.ShapeDtypeStruct((e, d), table.dtype),
                  jax.ShapeDtypeStruct((e, d), table.dtype)),
        mesh=mesh,
        scratch_types=[
            pltpu.VMEM((CH,), jnp.int32),
            pltpu.VMEM((CH,), jnp.int32),
            pltpu.VMEM((CH, d), table.dtype),
            pltpu.VMEM((CH, d), table.dtype),
            pltpu.SemaphoreType.DMA,
            pltpu.SemaphoreType.DMA,
        ],
    )
    def k(tab_h, ia_h, ib_h, oa_h, ob_h, ia_v, ib_v, ra_v, rb_v, sa, sb):
        cid = lax.axis_index("c")
        sid = lax.axis_index("s")
        wid = sid * 2 + cid
        nmine = nch // NW + jnp.where(wid < (nch % NW), 1, 0)

        def body(j, carry):
            off = (j * NW + wid) * CH
            pltpu.sync_copy(ia_h.at[pl.ds(ebase + off, CH)], ia_v)
            pltpu.sync_copy(ib_h.at[pl.ds(ebase + off, CH)], ib_v)
            ca = pltpu.async_copy(tab_h.at[ia_v], ra_v, sa)
            cb = pltpu.async_copy(tab_h.at[ib_v], rb_v, sb)
            ca.wait()
            pltpu.sync_copy(ra_v, oa_h.at[pl.ds(off, CH)])
            cb.wait()
            pltpu.sync_copy(rb_v, ob_h.at[pl.ds(off, CH)])
            return carry

        lax.fori_loop(0, nmine, body, 0)

    return k(table, ia, ib)


def _sc_scatter(vals, idx, ebase=0):
    e, d = vals.shape
    nch = e // CH
    rpt = N_PAD // 16
    mesh = plsc.VectorSubcoreMesh(core_axis_name="c", subcore_axis_name="s")

    @functools.partial(
        pl.kernel,
        out_type=jax.ShapeDtypeStruct((2, N_PAD, d), jnp.float32),
        mesh=mesh,
        scratch_types=[
            pltpu.VMEM((CH,), jnp.int32),
            pltpu.VMEM((CH, d), jnp.float32),
            pltpu.VMEM_SHARED((N_PAD, d), jnp.float32),
            pltpu.SemaphoreType.DMA,
        ],
    )
    def k(v_h, i_h, o_h, idx_v, rows_v, acc, sem):
        cid = lax.axis_index("c")
        sid = lax.axis_index("s")
        wid = sid * 2 + cid
        z16 = jnp.zeros((16,), jnp.float32)

        def zb(i, carry):
            for c0 in range(d // 16):
                rows_v[i, pl.ds(c0 * 16, 16)] = z16
            return carry

        lax.fori_loop(0, CH, zb, 0)
        tbase = sid * rpt

        def za(kk, carry):
            pltpu.sync_copy(rows_v, acc.at[pl.ds(tbase + kk * CH, CH)])
            return carry

        lax.fori_loop(0, rpt // CH, za, 0)
        plsc.subcore_barrier()

        nmine = nch // NW + jnp.where(wid < (nch % NW), 1, 0)

        def body(j, carry):
            off = (j * NW + wid) * CH
            pltpu.sync_copy(i_h.at[pl.ds(ebase + off, CH)], idx_v)
            pltpu.sync_copy(v_h.at[pl.ds(off, CH)], rows_v)
            pltpu.sync_copy(rows_v, acc.at[idx_v], add=True)
            return carry

        lax.fori_loop(0, nmine, body, 0)
        plsc.subcore_barrier()

        def wb(kk, carry):
            o = tbase + kk * CH
            pltpu.sync_copy(acc.at[pl.ds(o, CH)], o_h.at[cid, pl.ds(o, CH)])
            return carry

        lax.fori_loop(0, rpt // CH, wb, 0)

    return k(vals, idx)


def _sc_count(idx, e):
    d = 16
    nch = e // CH
    rpt = N_PAD // 16
    mesh = plsc.VectorSubcoreMesh(core_axis_name="c", subcore_axis_name="s")

    @functools.partial(
        pl.kernel,
        out_type=jax.ShapeDtypeStruct((2, N_PAD, d), jnp.float32),
        mesh=mesh,
        scratch_types=[
            pltpu.VMEM((CH,), jnp.int32),
            pltpu.VMEM((CH, d), jnp.float32),
            pltpu.VMEM((CH, d), jnp.float32),
            pltpu.VMEM_SHARED((N_PAD, d), jnp.float32),
            pltpu.SemaphoreType.DMA,
        ],
    )
    def k(i_h, o_h, idx_v, ones_v, zer_v, acc, sem):
        cid = lax.axis_index("c")
        sid = lax.axis_index("s")
        wid = sid * 2 + cid
        z16 = jnp.zeros((16,), jnp.float32)
        o16 = jnp.full((16,), 1.0, jnp.float32)

        def zb(i, carry):
            zer_v[i, pl.ds(0, 16)] = z16
            ones_v[i, pl.ds(0, 16)] = o16
            return carry

        lax.fori_loop(0, CH, zb, 0)
        tbase = sid * rpt

        def za(kk, carry):
            pltpu.sync_copy(zer_v, acc.at[pl.ds(tbase + kk * CH, CH)])
            return carry

        lax.fori_loop(0, rpt // CH, za, 0)
        plsc.subcore_barrier()

        nmine = nch // NW + jnp.where(wid < (nch % NW), 1, 0)

        def body(j, carry):
            off = (j * NW + wid) * CH
            pltpu.sync_copy(i_h.at[pl.ds(off, CH)], idx_v)
            pltpu.sync_copy(ones_v, acc.at[idx_v], add=True)
            return carry

        lax.fori_loop(0, nmine, body, 0)
        plsc.subcore_barrier()

        def wb(kk, carry):
            o = tbase + kk * CH
            pltpu.sync_copy(acc.at[pl.ds(o, CH)], o_h.at[cid, pl.ds(o, CH)])
            return carry

        lax.fori_loop(0, rpt // CH, wb, 0)

    return k(idx)


def _tc_embed(ids2d, emb_p):
    grid = N_N // TN

    def body(ids_ref, emb_ref, o_ref):
        ids = ids_ref[...]
        cols = lax.broadcasted_iota(jnp.int32, (TN, 16), 1)
        oh = (cols == ids).astype(jnp.float32)
        o_ref[...] = jnp.dot(oh, emb_ref[...],
                             preferred_element_type=jnp.float32)

    return pl.pallas_call(
        body,
        grid=(grid,),
        in_specs=[pl.BlockSpec((TN, 1), lambda i: (i, 0)),
                  pl.BlockSpec((16, DF), lambda i: (0, 0))],
        out_specs=pl.BlockSpec((TN, DF), lambda i: (i, 0)),
        out_shape=jax.ShapeDtypeStruct((N_N, DF), jnp.float32),
    )(ids2d, emb_p)


def _tc_fourier(csrc, cdst, inv_sc):
    grid = E_E // TE

    def body(a_ref, b_ref, s_ref, o_ref):
        rel = a_ref[...] - b_ref[...]
        rd = jnp.sum(rel * rel, axis=1, keepdims=True)
        xs = rd * s_ref[0:1, :]
        sn = jnp.sin(xs)
        cs = jnp.cos(xs)
        pad = jnp.zeros((TE, EAW - 65), jnp.float32)
        o_ref[...] = jnp.concatenate([sn, cs, rd, pad],
                                     axis=1).astype(jnp.bfloat16)

    return pl.pallas_call(
        body,
        grid=(grid,),
        in_specs=[pl.BlockSpec((TE, DF), lambda i: (i, 0)),
                  pl.BlockSpec((TE, DF), lambda i: (i, 0)),
                  pl.BlockSpec((8, 32), lambda i: (0, 0))],
        out_specs=pl.BlockSpec((TE, EAW), lambda i: (i, 0)),
        out_shape=jax.ShapeDtypeStruct((E_E, EAW), jnp.bfloat16),
    )(csrc, cdst, inv_sc)


def _tc_edge(xi, xj, ea, w1c, b1p, w2p, b2p, gp, bp, eab=0):
    e = xi.shape[0]
    grid = e // TEE

    def body(xi_ref, xj_ref, ea_ref, w1_ref, b1_ref,
             w2_ref, b2_ref, g_ref, b_ref, o_ref):
        x = jnp.concatenate(
            [xi_ref[...], xj_ref[...],
             ea_ref[...].astype(jnp.float32)], axis=1).astype(jnp.bfloat16)
        z = jnp.dot(x, w1_ref[...], preferred_element_type=jnp.float32)
        z = z + b1_ref[0:1, :]
        m = _silu(z).astype(jnp.bfloat16)
        y = jnp.dot(m, w2_ref[...], preferred_element_type=jnp.float32)
        y = _silu(y + b2_ref[0:1, :])
        o_ref[...] = _ln_rows(y, g_ref[0:1, :], b_ref[0:1, :])

    return pl.pallas_call(
        body,
        grid=(grid,),
        in_specs=[
            pl.BlockSpec((TEE, DF), lambda i: (i, 0)),
            pl.BlockSpec((TEE, DF), lambda i: (i, 0)),
            pl.BlockSpec((TEE, EAW), lambda i, b=eab: (i + b, 0)),
            pl.BlockSpec((2 * DF + EAW, H1), lambda i: (0, 0)),
            pl.BlockSpec((8, H1), lambda i: (0, 0)),
            pl.BlockSpec((H1, MM), lambda i: (0, 0)),
            pl.BlockSpec((8, MM), lambda i: (0, 0)),
            pl.BlockSpec((8, MM), lambda i: (0, 0)),
            pl.BlockSpec((8, MM), lambda i: (0, 0)),
        ],
        out_specs=pl.BlockSpec((TEE, MM), lambda i: (i, 0)),
        out_shape=jax.ShapeDtypeStruct((e, MM), jnp.float32),
    )(xi, xj, ea, w1c, b1p, w2p, b2p, gp, bp)


def _tc_node(feats, parts, parts2, cnts, w1h, w1m, nb1, w2t, nb2,
             eng, enb, n1g, n1b, n2g, n2b):
    grid = N_N // TN

    def body(f_ref, p_ref, p2_ref, c_ref, w1h_ref, w1m_ref, nb1_ref, w2t_ref,
             nb2_ref, eng_ref, enb_ref, n1g_ref, n1b_ref, n2g_ref,
             n2b_ref, o_ref):
        msum = p_ref[0] + p_ref[1] + p2_ref[0] + p2_ref[1]
        cnt = c_ref[0][:, 0:1] + c_ref[1][:, 0:1]
        m = msum / jnp.maximum(cnt, 1.0)
        m = _ln_rows(m, eng_ref[0:1, :], enb_ref[0:1, :])
        f = f_ref[...]
        h = _ln_rows(f, n1g_ref[0:1, :], n1b_ref[0:1, :])
        z = jnp.dot(h, w1h_ref[...], preferred_element_type=jnp.float32)
        z = z + jnp.dot(m, w1m_ref[...], preferred_element_type=jnp.float32)
        z = _silu(z + nb1_ref[0:1, :])
        h2 = jnp.dot(z, w2t_ref[...], preferred_element_type=jnp.float32)
        h2 = h2 + nb2_ref[0:1, :]
        h2 = _ln_rows(h2, n2g_ref[0:1, :], n2b_ref[0:1, :])
        o_ref[...] = f + h2

    return pl.pallas_call(
        body,
        grid=(grid,),
        in_specs=[
            pl.BlockSpec((TN, DF), lambda i: (i, 0)),
            pl.BlockSpec((2, TN, MM), lambda i: (0, i, 0)),
            pl.BlockSpec((2, TN, MM), lambda i: (0, i, 0)),
            pl.BlockSpec((2, TN, 16), lambda i: (0, i, 0)),
            pl.BlockSpec((DF, 2 * DF), lambda i: (0, 0)),
            pl.BlockSpec((MM, 2 * DF), lambda i: (0, 0)),
            pl.BlockSpec((8, 2 * DF), lambda i: (0, 0)),
            pl.BlockSpec((2 * DF, DF), lambda i: (0, 0)),
            pl.BlockSpec((8, DF), lambda i: (0, 0)),
            pl.BlockSpec((8, MM), lambda i: (0, 0)),
            pl.BlockSpec((8, MM), lambda i: (0, 0)),
            pl.BlockSpec((8, DF), lambda i: (0, 0)),
            pl.BlockSpec((8, DF), lambda i: (0, 0)),
            pl.BlockSpec((8, DF), lambda i: (0, 0)),
            pl.BlockSpec((8, DF), lambda i: (0, 0)),
        ],
        out_specs=pl.BlockSpec((TN, DF), lambda i: (i, 0)),
        out_shape=jax.ShapeDtypeStruct((N_N, DF), jnp.float32),
    )(feats, parts, parts2, cnts, w1h, w1m, nb1, w2t, nb2,
      eng, enb, n1g, n1b, n2g, n2b)


def _tc_readout(fl, batch3d, ws):
    (w1, b1, w2, b2, w3, b3, v1, c1, v2, c2, v3p, c3p) = ws
    grid = N_N // TN

    def body(f0_ref, f1_ref, f2_ref, f3_ref, f4_ref, f5_ref, bt_ref,
             w1_ref, b1_ref, w2_ref, b2_ref, w3_ref, b3_ref,
             v1_ref, c1_ref, v2_ref, c2_ref, v3_ref, c3_ref,
             o_ref, acc, cacc):
        i = pl.program_id(0)

        @pl.when(i == 0)
        def _():
            acc[...] = jnp.zeros_like(acc)
            cacc[...] = jnp.zeros_like(cacc)

        x = jnp.concatenate(
            [f0_ref[...], f1_ref[...], f2_ref[...],
             f3_ref[...], f4_ref[...], f5_ref[...]], axis=1)
        f = _silu(x)
        f = _silu(jnp.dot(f, w1_ref[...],
                          preferred_element_type=jnp.float32) + b1_ref[0:1, :])
        f = _silu(jnp.dot(f, w2_ref[...],
                          preferred_element_type=jnp.float32) + b2_ref[0:1, :])
        f = _silu(jnp.dot(f, w3_ref[...],
                          preferred_element_type=jnp.float32) + b3_ref[0:1, :])
        bt = bt_ref[0, 0, :]
        oh = (lax.broadcasted_iota(jnp.int32, (NB_G, TN), 0)
              == bt[None, :]).astype(jnp.float32)
        acc[...] = acc[...] + jnp.dot(oh, f,
                                      preferred_element_type=jnp.float32)
        cacc[...] = cacc[...] + jnp.sum(oh, axis=1, keepdims=True)

        @pl.when(i == grid - 1)
        def _():
            mean = acc[...] / jnp.maximum(cacc[:, 0:1], 1.0)
            g = _silu(jnp.dot(mean, v1_ref[...],
                              preferred_element_type=jnp.float32)
                      + c1_ref[0:1, :])
            g = _silu(jnp.dot(g, v2_ref[...],
                              preferred_element_type=jnp.float32)
                      + c2_ref[0:1, :])
            o_ref[...] = (jnp.dot(g, v3_ref[...],
                                  preferred_element_type=jnp.float32)
                          + c3_ref[0:1, :])

    return pl.pallas_call(
        body,
        grid=(grid,),
        in_specs=[
            pl.BlockSpec((TN, DF), lambda i: (i, 0)),
            pl.BlockSpec((TN, DF), lambda i: (i, 0)),
            pl.BlockSpec((TN, DF), lambda i: (i, 0)),
            pl.BlockSpec((TN, DF), lambda i: (i, 0)),
            pl.BlockSpec((TN, DF), lambda i: (i, 0)),
            pl.BlockSpec((TN, DF), lambda i: (i, 0)),
            pl.BlockSpec((1, 1, TN), lambda i: (i, 0, 0)),
            pl.BlockSpec((6 * DF, MLPD), lambda i: (0, 0)),
            pl.BlockSpec((8, MLPD), lambda i: (0, 0)),
            pl.BlockSpec((MLPD, MLPD), lambda i: (0, 0)),
            pl.BlockSpec((8, MLPD), lambda i: (0, 0)),
            pl.BlockSpec((MLPD, MLPD), lambda i: (0, 0)),
            pl.BlockSpec((8, MLPD), lambda i: (0, 0)),
            pl.BlockSpec((MLPD, MLPD), lambda i: (0, 0)),
            pl.BlockSpec((8, MLPD), lambda i: (0, 0)),
            pl.BlockSpec((MLPD, MLPD), lambda i: (0, 0)),
            pl.BlockSpec((8, MLPD), lambda i: (0, 0)),
            pl.BlockSpec((MLPD, DF), lambda i: (0, 0)),
            pl.BlockSpec((8, DF), lambda i: (0, 0)),
        ],
        out_specs=pl.BlockSpec((NB_G, DF), lambda i: (0, 0)),
        out_shape=jax.ShapeDtypeStruct((NB_G, DF), jnp.float32),
        scratch_shapes=[pltpu.VMEM((NB_G, MLPD), jnp.float32),
                        pltpu.VMEM((NB_G, DF), jnp.float32)],
    )(*fl, batch3d, w1, b1, w2, b2, w3, b3, v1, c1, v2, c2, v3p, c3p)


def _row8(v, width):
    out = jnp.zeros((8, width), jnp.float32)
    return out.at[0, :v.shape[0]].set(v)


def _prep_edge_weights(p):
    w1 = p["e_w1"]
    w1c = jnp.zeros((2 * DF + EAW, H1), jnp.float32).at[:321, :642].set(
        w1.T).astype(jnp.bfloat16)
    b1p = _row8(p["e_b1"], H1)
    w2p = jnp.zeros((H1, MM), jnp.float32).at[:642, :].set(
        p["e_w2"].T).astype(jnp.bfloat16)
    b2p = _row8(p["e_b2"], MM)
    gp = _row8(p["en_g"], MM)
    bp = _row8(p["en_b"], MM)
    return (w1c, b1p, w2p, b2p, gp, bp)


def _prep_node_weights(p):
    nw1 = p["n_w1"]
    w1h = nw1[:, :DF].T
    w1m = nw1[:, DF:].T
    nb1 = _row8(p["n_b1"], 2 * DF)
    w2t = p["n_w2"].T
    nb2 = _row8(p["n_b2"], DF)
    eng = _row8(p["en_g"], MM)
    enb = _row8(p["en_b"], MM)
    n1g = _row8(p["nn1_g"], DF)
    n1b = _row8(p["nn1_b"], DF)
    n2g = _row8(p["nn2_g"], DF)
    n2b = _row8(p["nn2_b"], DF)
    return (w1h, w1m, nb1, w2t, nb2, eng, enb, n1g, n1b, n2g, n2b)


def kernel(atomids, coords, edge_index, batch, params):
    src = edge_index[0].astype(jnp.int32)
    dst = edge_index[1].astype(jnp.int32)

    emb_p = jnp.zeros((16, DF), jnp.float32).at[:11].set(params["emb"])
    coords_p = jnp.zeros((N_N, DF), jnp.float32).at[:, :3].set(coords)
    inv_sc = jnp.zeros((8, 32), jnp.float32).at[0].set(
        jnp.exp2(-jnp.arange(32, dtype=jnp.float32)))

    csrc, cdst = _sc_gather2(coords_p, src, dst)
    ea = _tc_fourier(csrc, cdst, inv_sc)
    feats = _tc_embed(atomids.reshape(N_N, 1).astype(jnp.int32), emb_p)
    cnts = _sc_count(dst, E_E)

    fl = [feats]
    eh = E_E // 2
    for p in params["kernels"]:
        ew = _prep_edge_weights(p)
        nw = _prep_node_weights(p)
        xi0, xj0 = _sc_gather2(feats, dst, src, ebase=0, e=eh)
        m2a = _tc_edge(xi0, xj0, ea, *ew, eab=0)
        xi1, xj1 = _sc_gather2(feats, dst, src, ebase=eh, e=eh)
        m2b = _tc_edge(xi1, xj1, ea, *ew, eab=eh // TEE)
        pa = _sc_scatter(m2a, dst, ebase=0)
        pb = _sc_scatter(m2b, dst, ebase=eh)
        feats = _tc_node(feats, pa, pb, cnts, *nw)
        fl.append(feats)

    w1 = params["fnn"][0][0].T
    b1 = _row8(params["fnn"][0][1], MLPD)
    w2 = params["fnn"][1][0].T
    b2 = _row8(params["fnn"][1][1], MLPD)
    w3 = params["fnn"][2][0].T
    b3 = _row8(params["fnn"][2][1], MLPD)
    v1 = params["fnn2"][0][0].T
    c1 = _row8(params["fnn2"][0][1], MLPD)
    v2 = params["fnn2"][1][0].T
    c2 = _row8(params["fnn2"][1][1], MLPD)
    v3p = jnp.zeros((MLPD, DF), jnp.float32).at[:, 0].set(params["fnn2"][2][0][0])
    c3p = jnp.full((8, DF), params["fnn2"][2][1][0], jnp.float32)

    batch3d = batch.astype(jnp.int32).reshape(N_N // TN, 1, TN)
    out = _tc_readout(fl, batch3d,
                      (w1, b1, w2, b2, w3, b3, v1, c1, v2, c2, v3p, c3p))
    return out[:, :1]
